# Initial kernel scaffold; baseline (speedup 1.0000x reference)
import jax
import jax.numpy as jnp
from jax.experimental import pallas as pl


def kernel(x, table, W, b):
    raise NotImplementedError("write your pallas kernel here")


# speedup vs baseline: 7.3569x; 7.3569x over previous
import functools

import jax
import jax.numpy as jnp
from jax import lax
from jax.experimental import pallas as pl
from jax.experimental.pallas import tpu as pltpu
from jax.experimental.pallas import tpu_sc as plsc

VOCAB = 100000
EMBED = 300
NCLS = 6
BATCH = 4096
SEQ = 200

DPAD = 16
NCORES = 2
NSUB = 16
NW = NCORES * NSUB
ROWS_PER_W = BATCH // NW
CHUNK = 100
CPR = SEQ // CHUNK
NCHUNKS = ROWS_PER_W * CPR
GROUP = 8
NGROUPS = NCHUNKS // GROUP
ROWS_PER_GROUP = GROUP // CPR

VBLK = 2000


def _matmul_body(t_ref, w_ref, o_ref):
    o_ref[...] = jnp.dot(t_ref[...], w_ref[...],
                         preferred_element_type=jnp.float32)


def _table_times_w(table, wt):
    return pl.pallas_call(
        _matmul_body,
        grid=(VOCAB // VBLK,),
        in_specs=[
            pl.BlockSpec((VBLK, EMBED), lambda i: (i, 0)),
            pl.BlockSpec((EMBED, DPAD), lambda i: (0, 0)),
        ],
        out_specs=pl.BlockSpec((VBLK, DPAD), lambda i: (i, 0)),
        out_shape=jax.ShapeDtypeStruct((VOCAB, DPAD), jnp.float32),
    )(table, wt)


@functools.partial(
    pl.kernel,
    out_type=jax.ShapeDtypeStruct((BATCH, DPAD), jnp.float32),
    mesh=plsc.VectorSubcoreMesh(
        core_axis_name="c", subcore_axis_name="s",
        num_cores=NCORES, num_subcores=NSUB),
    scratch_types=[
        pltpu.VMEM((NCHUNKS, CHUNK), jnp.int32),
        pltpu.VMEM((2, GROUP, CHUNK, DPAD), jnp.float32),
        pltpu.VMEM((ROWS_PER_W, DPAD), jnp.float32),
        pltpu.VMEM((DPAD,), jnp.float32),
        pltpu.SemaphoreType.DMA,
        pltpu.SemaphoreType.DMA,
    ],
    compiler_params=pltpu.CompilerParams(use_tc_tiling_on_sc=False),
)
def _sc_pool(tw_hbm, x_hbm, bias_hbm, out_hbm,
             idx_v, gbuf, out_v, bias_v, sem0, sem1):
    wid = lax.axis_index("s") * NCORES + lax.axis_index("c")
    base = wid * ROWS_PER_W

    pltpu.sync_copy(x_hbm.at[pl.ds(base * CPR, NCHUNKS)], idx_v)
    pltpu.sync_copy(bias_hbm, bias_v)
    bias = bias_v[...]
    sems = (sem0, sem1)

    def fire(g, slot):
        for c in range(GROUP):
            pltpu.async_copy(tw_hbm.at[idx_v.at[g * GROUP + c]],
                             gbuf.at[slot, c], sems[slot])

    def drain_accum(g, slot):
        for c in range(GROUP):
            pltpu.make_async_copy(tw_hbm.at[idx_v.at[g * GROUP + c]],
                                  gbuf.at[slot, c], sems[slot]).wait()
        for q in range(ROWS_PER_GROUP):
            acc = jnp.zeros((DPAD,), jnp.float32)
            for half in range(CPR):
                chunk = gbuf.at[slot, q * CPR + half]

                def step(i, a, chunk=chunk):
                    j = i * 4
                    return a + ((chunk[j] + chunk[j + 1])
                                + (chunk[j + 2] + chunk[j + 3]))

                acc = lax.fori_loop(0, CHUNK // 4, step, acc)
            out_v[g * ROWS_PER_GROUP + q, :] = acc * (1.0 / SEQ) + bias

    fire(0, 0)

    def body(i, _):
        g = i * 2
        fire(g + 1, 1)
        drain_accum(g, 0)

        @pl.when(g + 2 < NGROUPS)
        def _():
            fire(g + 2, 0)

        drain_accum(g + 1, 1)
        return 0

    lax.fori_loop(0, NGROUPS // 2, body, 0)
    pltpu.sync_copy(out_v, out_hbm.at[pl.ds(base, ROWS_PER_W)])


def kernel(x, table, W, b):
    xi = x.astype(jnp.int32).reshape(BATCH * CPR, CHUNK)
    wt = jnp.zeros((EMBED, DPAD), jnp.float32).at[:, :NCLS].set(W.T)
    bias = jnp.zeros((DPAD,), jnp.float32).at[:NCLS].set(b)
    tw = _table_times_w(table, wt)
    out16 = _sc_pool(tw, xi, bias)
    return out16[:, :NCLS]


# speedup vs baseline: 12.9123x; 1.7551x over previous
import functools

import jax
import jax.numpy as jnp
from jax import lax
from jax.experimental import pallas as pl
from jax.experimental.pallas import tpu as pltpu
from jax.experimental.pallas import tpu_sc as plsc

VOCAB = 100000
EMBED = 300
NCLS = 6
BATCH = 4096
SEQ = 200

DPAD = 16
NCORES = 2
NSUB = 16
NW = NCORES * NSUB
ROWS_PER_W = BATCH // NW
CHUNK = 100
CPR = SEQ // CHUNK
NCHUNKS = ROWS_PER_W * CPR
GROUP = 8
NGROUPS = NCHUNKS // GROUP
ROWS_PER_GROUP = GROUP // CPR

VBLK = 8192


def _matmul_body(tt_ref, w_ref, o_ref):
    o_ref[...] = lax.dot_general(
        tt_ref[...], w_ref[...], (((0,), (0,)), ((), ())),
        preferred_element_type=jnp.float32)


def _table_times_w(table_t, wt):
    return pl.pallas_call(
        _matmul_body,
        grid=((VOCAB + VBLK - 1) // VBLK,),
        in_specs=[
            pl.BlockSpec((EMBED, VBLK), lambda i: (0, i)),
            pl.BlockSpec((EMBED, DPAD), lambda i: (0, 0)),
        ],
        out_specs=pl.BlockSpec((VBLK, DPAD), lambda i: (i, 0)),
        out_shape=jax.ShapeDtypeStruct((VOCAB, DPAD), jnp.float32),
    )(table_t, wt)


@functools.partial(
    pl.kernel,
    out_type=jax.ShapeDtypeStruct((BATCH, DPAD), jnp.float32),
    mesh=plsc.VectorSubcoreMesh(
        core_axis_name="c", subcore_axis_name="s",
        num_cores=NCORES, num_subcores=NSUB),
    scratch_types=[
        pltpu.VMEM((NCHUNKS, CHUNK), jnp.int32),
        pltpu.VMEM((2, GROUP, CHUNK, DPAD), jnp.float32),
        pltpu.VMEM((ROWS_PER_W, DPAD), jnp.float32),
        pltpu.VMEM((DPAD,), jnp.float32),
        pltpu.SemaphoreType.DMA,
        pltpu.SemaphoreType.DMA,
    ],
    compiler_params=pltpu.CompilerParams(use_tc_tiling_on_sc=False),
)
def _sc_pool(tw_hbm, x_hbm, bias_hbm, out_hbm,
             idx_v, gbuf, out_v, bias_v, sem0, sem1):
    wid = lax.axis_index("s") * NCORES + lax.axis_index("c")
    base = wid * ROWS_PER_W

    pltpu.sync_copy(x_hbm.at[pl.ds(base * CPR, NCHUNKS)], idx_v)
    pltpu.sync_copy(bias_hbm, bias_v)
    bias = bias_v[...]
    sems = (sem0, sem1)

    def fire(g, slot):
        for c in range(GROUP):
            pltpu.async_copy(tw_hbm.at[idx_v.at[g * GROUP + c]],
                             gbuf.at[slot, c], sems[slot])

    def drain_accum(g, slot):
        for c in range(GROUP):
            pltpu.make_async_copy(tw_hbm.at[idx_v.at[g * GROUP + c]],
                                  gbuf.at[slot, c], sems[slot]).wait()
        for q in range(ROWS_PER_GROUP):
            acc = jnp.zeros((DPAD,), jnp.float32)
            acc1 = jnp.zeros((DPAD,), jnp.float32)
            for half in range(CPR):
                chunk = gbuf.at[slot, q * CPR + half]
                for j in range(0, CHUNK, 4):
                    acc = acc + (chunk[j] + chunk[j + 1])
                    acc1 = acc1 + (chunk[j + 2] + chunk[j + 3])
            acc = acc + acc1
            out_v[g * ROWS_PER_GROUP + q, :] = acc * (1.0 / SEQ) + bias

    fire(0, 0)

    def body(i, _):
        g = i * 2
        fire(g + 1, 1)
        drain_accum(g, 0)

        @pl.when(g + 2 < NGROUPS)
        def _():
            fire(g + 2, 0)

        drain_accum(g + 1, 1)
        return 0

    lax.fori_loop(0, NGROUPS // 2, body, 0)
    pltpu.sync_copy(out_v, out_hbm.at[pl.ds(base, ROWS_PER_W)])


def kernel(x, table, W, b):
    xi = x.astype(jnp.int32).reshape(BATCH * CPR, CHUNK)
    wt = jnp.zeros((EMBED, DPAD), jnp.float32).at[:, :NCLS].set(W.T)
    bias = jnp.zeros((DPAD,), jnp.float32).at[:NCLS].set(b)
    tw = _table_times_w(table.T, wt)
    out16 = _sc_pool(tw, xi, bias)
    return out16[:, :NCLS]


# speedup vs baseline: 13.3028x; 1.0302x over previous
import functools

import jax
import jax.numpy as jnp
from jax import lax
from jax.experimental import pallas as pl
from jax.experimental.pallas import tpu as pltpu
from jax.experimental.pallas import tpu_sc as plsc

VOCAB = 100000
VOCAB_P = 100352
EMBED = 300
NCLS = 6
BATCH = 4096
SEQ = 200
SPAN = VOCAB_P // 32

DPAD = 16
NCORES = 2
NSUB = 16
NW = NCORES * NSUB
ROWS_PER_W = BATCH // NW
CHUNK = 100
CPR = SEQ // CHUNK
NCHUNKS = ROWS_PER_W * CPR
GROUP = 8
NGROUPS = NCHUNKS // GROUP
ROWS_PER_GROUP = GROUP // CPR

VBLK = 8192


def _matmul_body(tt_ref, w_ref, o_ref):
    o_ref[...] = lax.dot_general(
        w_ref[...], tt_ref[...], (((0,), (0,)), ((), ())),
        preferred_element_type=jnp.float32)


def _table_times_w(table_t, wt):
    return pl.pallas_call(
        _matmul_body,
        grid=((VOCAB_P + VBLK - 1) // VBLK,),
        in_specs=[
            pl.BlockSpec((EMBED, VBLK), lambda i: (0, i)),
            pl.BlockSpec((EMBED, DPAD), lambda i: (0, 0)),
        ],
        out_specs=pl.BlockSpec((DPAD, VBLK), lambda i: (0, i)),
        out_shape=jax.ShapeDtypeStruct((DPAD, VOCAB_P), jnp.float32),
    )(table_t, wt)


@functools.partial(
    pl.kernel,
    out_type=jax.ShapeDtypeStruct((VOCAB_P, DPAD), jnp.float32),
    mesh=plsc.VectorSubcoreMesh(
        core_axis_name="c", subcore_axis_name="s",
        num_cores=NCORES, num_subcores=NSUB),
    scratch_types=[
        pltpu.VMEM((DPAD, SPAN), jnp.float32),
        pltpu.VMEM((SPAN, DPAD), jnp.float32),
    ],
    compiler_params=pltpu.CompilerParams(use_tc_tiling_on_sc=False,
                                         needs_layout_passes=False),
)
def _sc_transpose(twt_hbm, out_hbm, buf, outb):
    wid = lax.axis_index("s") * NCORES + lax.axis_index("c")
    base = wid * SPAN
    pltpu.sync_copy(twt_hbm.at[:, pl.ds(base, SPAN)], buf)
    rows = lax.iota(jnp.int32, DPAD)

    @pl.loop(0, SPAN, unroll=8)
    def _(v):
        outb[v, :] = plsc.load_gather(buf, [rows, jnp.full((DPAD,), v,
                                                           jnp.int32)])

    pltpu.sync_copy(outb, out_hbm.at[pl.ds(base, SPAN)])


@functools.partial(
    pl.kernel,
    out_type=jax.ShapeDtypeStruct((BATCH, DPAD), jnp.float32),
    mesh=plsc.VectorSubcoreMesh(
        core_axis_name="c", subcore_axis_name="s",
        num_cores=NCORES, num_subcores=NSUB),
    scratch_types=[
        pltpu.VMEM((NCHUNKS, CHUNK), jnp.int32),
        pltpu.VMEM((2, GROUP, CHUNK, DPAD), jnp.float32),
        pltpu.VMEM((ROWS_PER_W, DPAD), jnp.float32),
        pltpu.VMEM((DPAD,), jnp.float32),
        pltpu.SemaphoreType.DMA,
        pltpu.SemaphoreType.DMA,
    ],
    compiler_params=pltpu.CompilerParams(use_tc_tiling_on_sc=False),
)
def _sc_pool(tw_hbm, x_hbm, bias_hbm, out_hbm,
             idx_v, gbuf, out_v, bias_v, sem0, sem1):
    wid = lax.axis_index("s") * NCORES + lax.axis_index("c")
    base = wid * ROWS_PER_W

    pltpu.sync_copy(x_hbm.at[pl.ds(base * CPR, NCHUNKS)], idx_v)
    pltpu.sync_copy(bias_hbm, bias_v)
    bias = bias_v[...]
    sems = (sem0, sem1)

    def fire(g, slot):
        for c in range(GROUP):
            pltpu.async_copy(tw_hbm.at[idx_v.at[g * GROUP + c]],
                             gbuf.at[slot, c], sems[slot])

    def drain_accum(g, slot):
        for c in range(GROUP):
            pltpu.make_async_copy(tw_hbm.at[idx_v.at[g * GROUP + c]],
                                  gbuf.at[slot, c], sems[slot]).wait()
        for q in range(ROWS_PER_GROUP):
            acc = jnp.zeros((DPAD,), jnp.float32)
            acc1 = jnp.zeros((DPAD,), jnp.float32)
            for half in range(CPR):
                chunk = gbuf.at[slot, q * CPR + half]
                for j in range(0, CHUNK, 4):
                    acc = acc + (chunk[j] + chunk[j + 1])
                    acc1 = acc1 + (chunk[j + 2] + chunk[j + 3])
            acc = acc + acc1
            out_v[g * ROWS_PER_GROUP + q, :] = acc * (1.0 / SEQ) + bias

    fire(0, 0)

    def body(i, _):
        g = i * 2
        fire(g + 1, 1)
        drain_accum(g, 0)

        @pl.when(g + 2 < NGROUPS)
        def _():
            fire(g + 2, 0)

        drain_accum(g + 1, 1)
        return 0

    lax.fori_loop(0, NGROUPS // 2, body, 0)
    pltpu.sync_copy(out_v, out_hbm.at[pl.ds(base, ROWS_PER_W)])


def kernel(x, table, W, b):
    xi = x.astype(jnp.int32).reshape(BATCH * CPR, CHUNK)
    wt = jnp.zeros((EMBED, DPAD), jnp.float32).at[:, :NCLS].set(W.T)
    bias = jnp.zeros((DPAD,), jnp.float32).at[:NCLS].set(b)
    tw_t = _table_times_w(table.T, wt)
    tw = _sc_transpose(tw_t)
    out16 = _sc_pool(tw, xi, bias)
    return out16[:, :NCLS]


# speedup vs baseline: 13.3215x; 1.0014x over previous
import functools

import jax
import jax.numpy as jnp
from jax import lax
from jax.experimental import pallas as pl
from jax.experimental.pallas import tpu as pltpu
from jax.experimental.pallas import tpu_sc as plsc

VOCAB = 100000
VOCAB_P = 100352
EMBED = 300
NCLS = 6
BATCH = 4096
SEQ = 200
SPAN = VOCAB_P // 32

DPAD = 16
NCORES = 2
NSUB = 16
NW = NCORES * NSUB
ROWS_PER_W = BATCH // NW
CHUNK = 100
CPR = SEQ // CHUNK
NCHUNKS = ROWS_PER_W * CPR
GROUP = 8
NGROUPS = NCHUNKS // GROUP
ROWS_PER_GROUP = GROUP // CPR

VBLK = 8192


def _matmul_body(tt_ref, w_ref, o_ref):
    o_ref[...] = lax.dot_general(
        w_ref[...], tt_ref[...], (((0,), (0,)), ((), ())),
        preferred_element_type=jnp.float32)


def _table_times_w(table_t, wt):
    return pl.pallas_call(
        _matmul_body,
        grid=((VOCAB_P + VBLK - 1) // VBLK,),
        in_specs=[
            pl.BlockSpec((EMBED, VBLK), lambda i: (0, i)),
            pl.BlockSpec((EMBED, DPAD), lambda i: (0, 0)),
        ],
        out_specs=pl.BlockSpec((DPAD, VBLK), lambda i: (0, i)),
        out_shape=jax.ShapeDtypeStruct((DPAD, VOCAB_P), jnp.float32),
    )(table_t, wt)


@functools.partial(
    pl.kernel,
    out_type=jax.ShapeDtypeStruct((VOCAB_P, DPAD), jnp.float32),
    mesh=plsc.VectorSubcoreMesh(
        core_axis_name="c", subcore_axis_name="s",
        num_cores=NCORES, num_subcores=NSUB),
    scratch_types=[
        pltpu.VMEM((DPAD * SPAN,), jnp.float32),
        pltpu.VMEM((SPAN, DPAD), jnp.float32),
        pltpu.SemaphoreType.DMA,
    ],
    compiler_params=pltpu.CompilerParams(use_tc_tiling_on_sc=False,
                                         needs_layout_passes=False),
)
def _sc_transpose(twt_hbm, out_hbm, buf, outb, sem):
    wid = lax.axis_index("s") * NCORES + lax.axis_index("c")
    base = wid * SPAN
    for r in range(DPAD):
        pltpu.async_copy(twt_hbm.at[r, pl.ds(base, SPAN)],
                         buf.at[pl.ds(r * SPAN, SPAN)], sem)
    for r in range(DPAD):
        pltpu.make_async_copy(twt_hbm.at[r, pl.ds(base, SPAN)],
                              buf.at[pl.ds(r * SPAN, SPAN)], sem).wait()
    colbase = lax.iota(jnp.int32, 16) * SPAN

    def body(i, _):
        for j in range(16):
            v = i * 16 + j
            outb[v, :] = plsc.load_gather(buf, [colbase + v])
        return 0

    lax.fori_loop(0, SPAN // 16, body, 0)
    pltpu.sync_copy(outb, out_hbm.at[pl.ds(base, SPAN)])


@functools.partial(
    pl.kernel,
    out_type=jax.ShapeDtypeStruct((BATCH, DPAD), jnp.float32),
    mesh=plsc.VectorSubcoreMesh(
        core_axis_name="c", subcore_axis_name="s",
        num_cores=NCORES, num_subcores=NSUB),
    scratch_types=[
        pltpu.VMEM((NCHUNKS, CHUNK), jnp.int32),
        pltpu.VMEM((2, GROUP, CHUNK, DPAD), jnp.float32),
        pltpu.VMEM((ROWS_PER_W, DPAD), jnp.float32),
        pltpu.VMEM((DPAD,), jnp.float32),
        pltpu.SemaphoreType.DMA,
        pltpu.SemaphoreType.DMA,
    ],
    compiler_params=pltpu.CompilerParams(use_tc_tiling_on_sc=False),
)
def _sc_pool(tw_hbm, x_hbm, bias_hbm, out_hbm,
             idx_v, gbuf, out_v, bias_v, sem0, sem1):
    wid = lax.axis_index("s") * NCORES + lax.axis_index("c")
    base = wid * ROWS_PER_W

    pltpu.sync_copy(x_hbm.at[pl.ds(base * CPR, NCHUNKS)], idx_v)
    pltpu.sync_copy(bias_hbm, bias_v)
    bias = bias_v[...]
    sems = (sem0, sem1)

    def fire(g, slot):
        for c in range(GROUP):
            pltpu.async_copy(tw_hbm.at[idx_v.at[g * GROUP + c]],
                             gbuf.at[slot, c], sems[slot])

    def drain_accum(g, slot):
        for c in range(GROUP):
            pltpu.make_async_copy(tw_hbm.at[idx_v.at[g * GROUP + c]],
                                  gbuf.at[slot, c], sems[slot]).wait()
        for q in range(ROWS_PER_GROUP):
            acc = jnp.zeros((DPAD,), jnp.float32)
            acc1 = jnp.zeros((DPAD,), jnp.float32)
            for half in range(CPR):
                chunk = gbuf.at[slot, q * CPR + half]
                for j in range(0, CHUNK, 4):
                    acc = acc + (chunk[j] + chunk[j + 1])
                    acc1 = acc1 + (chunk[j + 2] + chunk[j + 3])
            acc = acc + acc1
            out_v[g * ROWS_PER_GROUP + q, :] = acc * (1.0 / SEQ) + bias

    fire(0, 0)

    def body(i, _):
        g = i * 2
        fire(g + 1, 1)
        drain_accum(g, 0)

        @pl.when(g + 2 < NGROUPS)
        def _():
            fire(g + 2, 0)

        drain_accum(g + 1, 1)
        return 0

    lax.fori_loop(0, NGROUPS // 2, body, 0)
    pltpu.sync_copy(out_v, out_hbm.at[pl.ds(base, ROWS_PER_W)])


def kernel(x, table, W, b):
    xi = x.astype(jnp.int32).reshape(BATCH * CPR, CHUNK)
    wt = jnp.zeros((EMBED, DPAD), jnp.float32).at[:, :NCLS].set(W.T)
    bias = jnp.zeros((DPAD,), jnp.float32).at[:NCLS].set(b)
    tw_t = _table_times_w(table.T, wt)
    tw = _sc_transpose(tw_t)
    out16 = _sc_pool(tw, xi, bias)
    return out16[:, :NCLS]


# speedup vs baseline: 17.5634x; 1.3184x over previous
import functools

import jax
import jax.numpy as jnp
from jax import lax
from jax.experimental import pallas as pl
from jax.experimental.pallas import tpu as pltpu
from jax.experimental.pallas import tpu_sc as plsc

VOCAB = 100000
VOCAB_P = 100352
EMBED = 300
NCLS = 6
BATCH = 4096
SEQ = 200
SPAN = VOCAB_P // 32

DPAD = 8
OSTRIDE = DPAD + 1
NCORES = 2
NSUB = 16
NW = NCORES * NSUB
ROWS_PER_W = BATCH // NW
CHUNK = 100
CPR = SEQ // CHUNK
NCHUNKS = ROWS_PER_W * CPR
GROUP = 8
NGROUPS = NCHUNKS // GROUP
ROWS_PER_GROUP = GROUP // CPR

VBLK = 8192


def _matmul_body(tt_ref, w_ref, o_ref):
    o_ref[...] = lax.dot_general(
        w_ref[...], tt_ref[...], (((0,), (0,)), ((), ())),
        preferred_element_type=jnp.float32)


def _table_times_w(table_t, wt):
    return pl.pallas_call(
        _matmul_body,
        grid=((VOCAB_P + VBLK - 1) // VBLK,),
        in_specs=[
            pl.BlockSpec((EMBED, VBLK), lambda i: (0, i)),
            pl.BlockSpec((EMBED, DPAD), lambda i: (0, 0)),
        ],
        out_specs=pl.BlockSpec((DPAD, VBLK), lambda i: (0, i)),
        out_shape=jax.ShapeDtypeStruct((DPAD, VOCAB_P), jnp.float32),
    )(table_t, wt)


@functools.partial(
    pl.kernel,
    out_type=jax.ShapeDtypeStruct((VOCAB_P, DPAD), jnp.float32),
    mesh=plsc.VectorSubcoreMesh(
        core_axis_name="c", subcore_axis_name="s",
        num_cores=NCORES, num_subcores=NSUB),
    scratch_types=[
        pltpu.VMEM((DPAD, SPAN), jnp.float32),
        pltpu.VMEM((SPAN, OSTRIDE), jnp.float32),
    ],
    compiler_params=pltpu.CompilerParams(use_tc_tiling_on_sc=False,
                                         needs_layout_passes=False),
)
def _sc_transpose(twt_hbm, out_hbm, buf, outb):
    wid = lax.axis_index("s") * NCORES + lax.axis_index("c")
    base = wid * SPAN
    pltpu.sync_copy(twt_hbm.at[:, pl.ds(base, SPAN)], buf)
    lanes = lax.iota(jnp.int32, 16)
    cols = [jnp.full((16,), r, jnp.int32) for r in range(DPAD)]

    def body(i, _):
        v0 = i * 16
        vrow = lanes + v0
        for r in range(DPAD):
            plsc.store_scatter(outb, [vrow, cols[r]],
                               buf[r, pl.ds(v0, 16)])
        return 0

    lax.fori_loop(0, SPAN // 16, body, 0)
    pltpu.sync_copy(outb.at[:, pl.ds(0, DPAD)],
                    out_hbm.at[pl.ds(base, SPAN)])


@functools.partial(
    pl.kernel,
    out_type=jax.ShapeDtypeStruct((BATCH, 16), jnp.float32),
    mesh=plsc.VectorSubcoreMesh(
        core_axis_name="c", subcore_axis_name="s",
        num_cores=NCORES, num_subcores=NSUB),
    scratch_types=[
        pltpu.VMEM((NCHUNKS, CHUNK), jnp.int32),
        pltpu.VMEM((2, GROUP, CHUNK, DPAD), jnp.float32),
        pltpu.VMEM((ROWS_PER_W, 16), jnp.float32),
        pltpu.VMEM((16,), jnp.float32),
        pltpu.VMEM((16,), jnp.float32),
        pltpu.SemaphoreType.DMA,
        pltpu.SemaphoreType.DMA,
    ],
    compiler_params=pltpu.CompilerParams(use_tc_tiling_on_sc=False,
                                         needs_layout_passes=False),
)
def _sc_pool(tw_hbm, x_hbm, bias_hbm, out_hbm,
             idx_v, gbuf, out_v, bias_v, scr, sem0, sem1):
    wid = lax.axis_index("s") * NCORES + lax.axis_index("c")
    base = wid * ROWS_PER_W

    pltpu.sync_copy(x_hbm.at[pl.ds(base * CPR, NCHUNKS)], idx_v)
    pltpu.sync_copy(bias_hbm, bias_v)
    bias = bias_v[...]
    sems = (sem0, sem1)
    lanes = lax.iota(jnp.int32, 16)
    row2 = lanes // 8
    col8 = lanes % 8
    shift = (lanes + 8) % 16

    def fire(g, slot):
        for c in range(GROUP):
            pltpu.async_copy(tw_hbm.at[idx_v.at[g * GROUP + c]],
                             gbuf.at[slot, c], sems[slot])

    def drain_accum(g, slot):
        for c in range(GROUP):
            pltpu.make_async_copy(tw_hbm.at[idx_v.at[g * GROUP + c]],
                                  gbuf.at[slot, c], sems[slot]).wait()
        for q in range(ROWS_PER_GROUP):
            acc = jnp.zeros((16,), jnp.float32)
            acc1 = jnp.zeros((16,), jnp.float32)
            for half in range(CPR):
                chunk = gbuf.at[slot, q * CPR + half]
                idxr = row2
                idxr1 = row2 + 2
                for _ in range(CHUNK // 4):
                    acc = acc + plsc.load_gather(chunk, [idxr, col8])
                    acc1 = acc1 + plsc.load_gather(chunk, [idxr1, col8])
                    idxr = idxr + 4
                    idxr1 = idxr1 + 4
            acc = acc + acc1
            scr[...] = acc
            acc = acc + plsc.load_gather(scr, [shift])
            out_v[g * ROWS_PER_GROUP + q, :] = acc * (1.0 / SEQ) + bias

    fire(0, 0)

    def body(i, _):
        g = i * 2
        fire(g + 1, 1)
        drain_accum(g, 0)

        @pl.when(g + 2 < NGROUPS)
        def _():
            fire(g + 2, 0)

        drain_accum(g + 1, 1)
        return 0

    lax.fori_loop(0, NGROUPS // 2, body, 0)
    pltpu.sync_copy(out_v, out_hbm.at[pl.ds(base, ROWS_PER_W)])


def kernel(x, table, W, b):
    xi = x.astype(jnp.int32).reshape(BATCH * CPR, CHUNK)
    wt = jnp.zeros((EMBED, DPAD), jnp.float32).at[:, :NCLS].set(W.T)
    bias = (jnp.zeros((16,), jnp.float32)
            .at[:NCLS].set(b).at[8:8 + NCLS].set(b))
    tw_t = _table_times_w(table.T, wt)
    tw = _sc_transpose(tw_t)
    out16 = _sc_pool(tw, xi, bias)
    return out16[:, :NCLS]


# speedup vs baseline: 17.6168x; 1.0030x over previous
import functools

import jax
import jax.numpy as jnp
from jax import lax
from jax.experimental import pallas as pl
from jax.experimental.pallas import tpu as pltpu
from jax.experimental.pallas import tpu_sc as plsc

VOCAB = 100000
VOCAB_P = 100352
EMBED = 300
NCLS = 6
BATCH = 4096
SEQ = 200
SPAN = VOCAB_P // 32

DPAD = 8
OSTRIDE = DPAD + 1
NCORES = 2
NSUB = 16
NW = NCORES * NSUB
COLS_PER_W = BATCH // NW
PAIRS = COLS_PER_W // 2
LC = 20
NG = SEQ // LC

VBLK = 8192


def _matmul_body(tt_ref, w_ref, o_ref):
    o_ref[...] = lax.dot_general(
        w_ref[...], tt_ref[...], (((0,), (0,)), ((), ())),
        preferred_element_type=jnp.float32)


def _table_times_w(table_t, wt):
    return pl.pallas_call(
        _matmul_body,
        grid=((VOCAB_P + VBLK - 1) // VBLK,),
        in_specs=[
            pl.BlockSpec((EMBED, VBLK), lambda i: (0, i)),
            pl.BlockSpec((EMBED, DPAD), lambda i: (0, 0)),
        ],
        out_specs=pl.BlockSpec((DPAD, VBLK), lambda i: (0, i)),
        out_shape=jax.ShapeDtypeStruct((DPAD, VOCAB_P), jnp.float32),
    )(table_t, wt)


@functools.partial(
    pl.kernel,
    out_type=jax.ShapeDtypeStruct((VOCAB_P, DPAD), jnp.float32),
    mesh=plsc.VectorSubcoreMesh(
        core_axis_name="c", subcore_axis_name="s",
        num_cores=NCORES, num_subcores=NSUB),
    scratch_types=[
        pltpu.VMEM((DPAD, SPAN), jnp.float32),
        pltpu.VMEM((SPAN, OSTRIDE), jnp.float32),
    ],
    compiler_params=pltpu.CompilerParams(use_tc_tiling_on_sc=False,
                                         needs_layout_passes=False),
)
def _sc_transpose(twt_hbm, out_hbm, buf, outb):
    wid = lax.axis_index("s") * NCORES + lax.axis_index("c")
    base = wid * SPAN
    pltpu.sync_copy(twt_hbm.at[:, pl.ds(base, SPAN)], buf)
    lanes = lax.iota(jnp.int32, 16)
    cols = [jnp.full((16,), r, jnp.int32) for r in range(DPAD)]

    def body(i, _):
        v0 = i * 16
        vrow = lanes + v0
        for r in range(DPAD):
            plsc.store_scatter(outb, [vrow, cols[r]],
                               buf[r, pl.ds(v0, 16)])
        return 0

    lax.fori_loop(0, SPAN // 16, body, 0)
    pltpu.sync_copy(outb.at[:, pl.ds(0, DPAD)],
                    out_hbm.at[pl.ds(base, SPAN)])


@functools.partial(
    pl.kernel,
    out_type=jax.ShapeDtypeStruct((BATCH * DPAD,), jnp.float32),
    mesh=plsc.VectorSubcoreMesh(
        core_axis_name="c", subcore_axis_name="s",
        num_cores=NCORES, num_subcores=NSUB),
    scratch_types=[
        pltpu.VMEM((SEQ, COLS_PER_W), jnp.int32),
        pltpu.VMEM((2, LC, COLS_PER_W, DPAD), jnp.float32),
        pltpu.VMEM((COLS_PER_W * DPAD,), jnp.float32),
        pltpu.VMEM((16,), jnp.float32),
        pltpu.SemaphoreType.DMA,
        pltpu.SemaphoreType.DMA,
    ],
    compiler_params=pltpu.CompilerParams(use_tc_tiling_on_sc=False,
                                         needs_layout_passes=False),
)
def _sc_pool(tw_hbm, xt_hbm, bias_hbm, out_hbm,
             idx_v, gbuf, out_v, bias_v, sem0, sem1):
    wid = lax.axis_index("s") * NCORES + lax.axis_index("c")
    base = wid * COLS_PER_W

    pltpu.sync_copy(xt_hbm.at[:, pl.ds(base, COLS_PER_W)], idx_v)
    pltpu.sync_copy(bias_hbm, bias_v)
    bias = bias_v[...]
    sems = (sem0, sem1)
    lanes = lax.iota(jnp.int32, 16)
    row2 = lanes // 8
    col8 = lanes % 8
    zero16 = jnp.zeros((16,), jnp.float32)
    for t in range(PAIRS):
        out_v[pl.ds(t * 16, 16)] = zero16

    def fire(g, slot):
        for c in range(LC):
            pltpu.async_copy(tw_hbm.at[idx_v.at[g * LC + c]],
                             gbuf.at[slot, c], sems[slot])

    def drain_accum(g, slot):
        for c in range(LC):
            pltpu.make_async_copy(tw_hbm.at[idx_v.at[g * LC + c]],
                                  gbuf.at[slot, c], sems[slot]).wait()
        for bg in range(PAIRS // 8):
            rows = [row2 + (bg * 16 + 2 * p) for p in range(8)]

            def step(c, accs, rows=rows, slot=slot):
                chunk = gbuf.at[slot, c]
                return tuple(
                    accs[p] + plsc.load_gather(chunk, [rows[p], col8])
                    for p in range(8))

            accs = lax.fori_loop(0, LC, step, tuple(zero16
                                                    for _ in range(8)))
            for p in range(8):
                plsc.addupdate(out_v.at[pl.ds((bg * 8 + p) * 16, 16)],
                               accs[p])

    fire(0, 0)

    def body(i, _):
        g = i * 2
        fire(g + 1, 1)
        drain_accum(g, 0)

        @pl.when(g + 2 < NG)
        def _():
            fire(g + 2, 0)

        drain_accum(g + 1, 1)
        return 0

    lax.fori_loop(0, NG // 2, body, 0)
    for t in range(PAIRS):
        v = out_v[pl.ds(t * 16, 16)]
        out_v[pl.ds(t * 16, 16)] = v * (1.0 / SEQ) + bias
    pltpu.sync_copy(out_v,
                    out_hbm.at[pl.ds(base * DPAD, COLS_PER_W * DPAD)])


def kernel(x, table, W, b):
    xt = x.astype(jnp.int32).T
    wt = jnp.zeros((EMBED, DPAD), jnp.float32).at[:, :NCLS].set(W.T)
    bias = (jnp.zeros((16,), jnp.float32)
            .at[:NCLS].set(b).at[8:8 + NCLS].set(b))
    tw_t = _table_times_w(table.T, wt)
    tw = _sc_transpose(tw_t)
    out = _sc_pool(tw, xt, bias)
    return out.reshape(BATCH, DPAD)[:, :NCLS]


# speedup vs baseline: 18.6399x; 1.0581x over previous
import functools

import jax
import jax.numpy as jnp
from jax import lax
from jax.experimental import pallas as pl
from jax.experimental.pallas import tpu as pltpu
from jax.experimental.pallas import tpu_sc as plsc

VOCAB = 100000
VOCAB_P = 100352
EMBED = 300
NCLS = 6
BATCH = 4096
SEQ = 200
SPAN = VOCAB_P // 32

DPAD = 8
OSTRIDE = DPAD + 1
NCORES = 2
NSUB = 16
NW = NCORES * NSUB
COLS_PER_W = BATCH // NW
PAIRS = COLS_PER_W // 2
LC = 20
NG = SEQ // LC

VBLK = 8192


def _matmul_body(tt_ref, w_ref, o_ref):
    o_ref[...] = lax.dot_general(
        w_ref[...], tt_ref[...], (((0,), (0,)), ((), ())),
        preferred_element_type=jnp.float32)


def _table_times_w(table_t, wt):
    return pl.pallas_call(
        _matmul_body,
        grid=((VOCAB_P + VBLK - 1) // VBLK,),
        in_specs=[
            pl.BlockSpec((EMBED, VBLK), lambda i: (0, i)),
            pl.BlockSpec((EMBED, DPAD), lambda i: (0, 0)),
        ],
        out_specs=pl.BlockSpec((DPAD, VBLK), lambda i: (0, i)),
        out_shape=jax.ShapeDtypeStruct((DPAD, VOCAB_P), jnp.float32),
    )(table_t, wt)


@functools.partial(
    pl.kernel,
    out_type=jax.ShapeDtypeStruct((VOCAB_P * DPAD,), jnp.float32),
    mesh=plsc.VectorSubcoreMesh(
        core_axis_name="c", subcore_axis_name="s",
        num_cores=NCORES, num_subcores=NSUB),
    scratch_types=[
        pltpu.VMEM((DPAD, SPAN), jnp.float32),
        pltpu.VMEM((SPAN * OSTRIDE,), jnp.float32),
        pltpu.VMEM((SPAN * DPAD,), jnp.float32),
    ],
    compiler_params=pltpu.CompilerParams(use_tc_tiling_on_sc=False,
                                         needs_layout_passes=False),
)
def _sc_transpose(twt_hbm, out_hbm, buf, outb, outc):
    wid = lax.axis_index("s") * NCORES + lax.axis_index("c")
    base = wid * SPAN
    pltpu.sync_copy(twt_hbm.at[:, pl.ds(base, SPAN)], buf)
    lanes = lax.iota(jnp.int32, 16)
    v9 = lanes * OSTRIDE

    def body(i, _):
        b9 = v9 + i * (16 * OSTRIDE)
        for r in range(DPAD):
            plsc.store_scatter(outb, [b9 + r], buf[r, pl.ds(i * 16, 16)])
        return 0

    lax.fori_loop(0, SPAN // 16, body, 0)

    def compact(k, _):
        t = lanes + k * 16
        outc[pl.ds(k * 16, 16)] = plsc.load_gather(outb, [t + t // DPAD])
        return 0

    lax.fori_loop(0, SPAN * DPAD // 16, compact, 0)
    pltpu.sync_copy(outc, out_hbm.at[pl.ds(base * DPAD, SPAN * DPAD)])


@functools.partial(
    pl.kernel,
    out_type=jax.ShapeDtypeStruct((BATCH * DPAD,), jnp.float32),
    mesh=plsc.VectorSubcoreMesh(
        core_axis_name="c", subcore_axis_name="s",
        num_cores=NCORES, num_subcores=NSUB),
    scratch_types=[
        pltpu.VMEM((SEQ, COLS_PER_W), jnp.int32),
        pltpu.VMEM((2, LC, COLS_PER_W, DPAD), jnp.float32),
        pltpu.VMEM((COLS_PER_W * DPAD,), jnp.float32),
        pltpu.VMEM((16,), jnp.float32),
        pltpu.SemaphoreType.DMA,
        pltpu.SemaphoreType.DMA,
    ],
    compiler_params=pltpu.CompilerParams(use_tc_tiling_on_sc=False,
                                         needs_layout_passes=False),
)
def _sc_pool(tw_hbm, xt_hbm, bias_hbm, out_hbm,
             idx_v, gbuf, out_v, bias_v, sem0, sem1):
    wid = lax.axis_index("s") * NCORES + lax.axis_index("c")
    base = wid * COLS_PER_W

    pltpu.sync_copy(xt_hbm.at[:, pl.ds(base, COLS_PER_W)], idx_v)
    pltpu.sync_copy(bias_hbm, bias_v)
    bias = bias_v[...]
    sems = (sem0, sem1)
    lanes = lax.iota(jnp.int32, 16)
    row2 = lanes // 8
    col8 = lanes % 8
    zero16 = jnp.zeros((16,), jnp.float32)
    for t in range(PAIRS):
        out_v[pl.ds(t * 16, 16)] = zero16

    def fire(g, slot):
        for c in range(LC):
            pltpu.async_copy(tw_hbm.at[idx_v.at[g * LC + c]],
                             gbuf.at[slot, c], sems[slot])

    def drain_accum(g, slot):
        for c in range(LC):
            pltpu.make_async_copy(tw_hbm.at[idx_v.at[g * LC + c]],
                                  gbuf.at[slot, c], sems[slot]).wait()
        for bg in range(PAIRS // 8):
            rows = [row2 + (bg * 16 + 2 * p) for p in range(8)]

            def step(c, accs, rows=rows, slot=slot):
                chunk = gbuf.at[slot, c]
                return tuple(
                    accs[p] + plsc.load_gather(chunk, [rows[p], col8])
                    for p in range(8))

            accs = lax.fori_loop(0, LC, step, tuple(zero16
                                                    for _ in range(8)))
            for p in range(8):
                plsc.addupdate(out_v.at[pl.ds((bg * 8 + p) * 16, 16)],
                               accs[p])

    fire(0, 0)

    def body(i, _):
        g = i * 2
        fire(g + 1, 1)
        drain_accum(g, 0)

        @pl.when(g + 2 < NG)
        def _():
            fire(g + 2, 0)

        drain_accum(g + 1, 1)
        return 0

    lax.fori_loop(0, NG // 2, body, 0)
    for t in range(PAIRS):
        v = out_v[pl.ds(t * 16, 16)]
        out_v[pl.ds(t * 16, 16)] = v * (1.0 / SEQ) + bias
    pltpu.sync_copy(out_v,
                    out_hbm.at[pl.ds(base * DPAD, COLS_PER_W * DPAD)])


def kernel(x, table, W, b):
    xt = x.astype(jnp.int32).T
    wt = jnp.zeros((EMBED, DPAD), jnp.float32).at[:, :NCLS].set(W.T)
    bias = (jnp.zeros((16,), jnp.float32)
            .at[:NCLS].set(b).at[8:8 + NCLS].set(b))
    tw_t = _table_times_w(table.T, wt)
    tw = _sc_transpose(tw_t).reshape(VOCAB_P, DPAD)
    out = _sc_pool(tw, xt, bias)
    return out.reshape(BATCH, DPAD)[:, :NCLS]


# speedup vs baseline: 18.6789x; 1.0021x over previous
import functools

import jax
import jax.numpy as jnp
from jax import lax
from jax.experimental import pallas as pl
from jax.experimental.pallas import tpu as pltpu
from jax.experimental.pallas import tpu_sc as plsc

VOCAB = 100000
VOCAB_P = 100352
EMBED = 300
NCLS = 6
BATCH = 4096
SEQ = 200
SPAN = VOCAB_P // 32

DPAD = 8
OSTRIDE = DPAD + 1
NCORES = 2
NSUB = 16
NW = NCORES * NSUB
COLS_PER_W = BATCH // NW
PAIRS = COLS_PER_W // 2
LC = 25
NG = SEQ // LC

VBLK = 8192


def _matmul_body(tt_ref, w_ref, o_ref):
    o_ref[...] = lax.dot_general(
        w_ref[...], tt_ref[...], (((0,), (0,)), ((), ())),
        preferred_element_type=jnp.float32)


def _table_times_w(table_t, wt):
    return pl.pallas_call(
        _matmul_body,
        grid=((VOCAB_P + VBLK - 1) // VBLK,),
        in_specs=[
            pl.BlockSpec((EMBED, VBLK), lambda i: (0, i)),
            pl.BlockSpec((EMBED, DPAD), lambda i: (0, 0)),
        ],
        out_specs=pl.BlockSpec((DPAD, VBLK), lambda i: (0, i)),
        out_shape=jax.ShapeDtypeStruct((DPAD, VOCAB_P), jnp.float32),
    )(table_t, wt)


@functools.partial(
    pl.kernel,
    out_type=jax.ShapeDtypeStruct((VOCAB_P * DPAD,), jnp.float32),
    mesh=plsc.VectorSubcoreMesh(
        core_axis_name="c", subcore_axis_name="s",
        num_cores=NCORES, num_subcores=NSUB),
    scratch_types=[
        pltpu.VMEM((DPAD, SPAN), jnp.float32),
        pltpu.VMEM((SPAN * OSTRIDE,), jnp.float32),
        pltpu.VMEM((SPAN * DPAD,), jnp.float32),
    ],
    compiler_params=pltpu.CompilerParams(use_tc_tiling_on_sc=False,
                                         needs_layout_passes=False),
)
def _sc_transpose(twt_hbm, out_hbm, buf, outb, outc):
    wid = lax.axis_index("s") * NCORES + lax.axis_index("c")
    base = wid * SPAN
    pltpu.sync_copy(twt_hbm.at[:, pl.ds(base, SPAN)], buf)
    lanes = lax.iota(jnp.int32, 16)
    v9 = lanes * OSTRIDE

    def body(i, _):
        b9 = v9 + i * (16 * OSTRIDE)
        for r in range(DPAD):
            plsc.store_scatter(outb, [b9 + r], buf[r, pl.ds(i * 16, 16)])
        return 0

    lax.fori_loop(0, SPAN // 16, body, 0)

    def compact(k, _):
        t = lanes + k * 16
        outc[pl.ds(k * 16, 16)] = plsc.load_gather(outb, [t + t // DPAD])
        return 0

    lax.fori_loop(0, SPAN * DPAD // 16, compact, 0)
    pltpu.sync_copy(outc, out_hbm.at[pl.ds(base * DPAD, SPAN * DPAD)])


@functools.partial(
    pl.kernel,
    out_type=jax.ShapeDtypeStruct((BATCH * DPAD,), jnp.float32),
    mesh=plsc.VectorSubcoreMesh(
        core_axis_name="c", subcore_axis_name="s",
        num_cores=NCORES, num_subcores=NSUB),
    scratch_types=[
        pltpu.VMEM((SEQ, COLS_PER_W), jnp.int32),
        pltpu.VMEM((2, LC, COLS_PER_W, DPAD), jnp.float32),
        pltpu.VMEM((COLS_PER_W * DPAD,), jnp.float32),
        pltpu.VMEM((16,), jnp.float32),
        pltpu.SemaphoreType.DMA,
        pltpu.SemaphoreType.DMA,
    ],
    compiler_params=pltpu.CompilerParams(use_tc_tiling_on_sc=False,
                                         needs_layout_passes=False),
)
def _sc_pool(tw_hbm, xt_hbm, bias_hbm, out_hbm,
             idx_v, gbuf, out_v, bias_v, sem0, sem1):
    wid = lax.axis_index("s") * NCORES + lax.axis_index("c")
    base = wid * COLS_PER_W

    pltpu.sync_copy(xt_hbm.at[:, pl.ds(base, COLS_PER_W)], idx_v)
    pltpu.sync_copy(bias_hbm, bias_v)
    bias = bias_v[...]
    sems = (sem0, sem1)
    lanes = lax.iota(jnp.int32, 16)
    row2 = lanes // 8
    col8 = lanes % 8
    zero16 = jnp.zeros((16,), jnp.float32)
    for t in range(PAIRS):
        out_v[pl.ds(t * 16, 16)] = zero16

    def fire(g, slot):
        for c in range(LC):
            pltpu.async_copy(tw_hbm.at[idx_v.at[g * LC + c]],
                             gbuf.at[slot, c], sems[slot])

    def drain_accum(g, slot):
        for c in range(LC):
            pltpu.make_async_copy(tw_hbm.at[idx_v.at[g * LC + c]],
                                  gbuf.at[slot, c], sems[slot]).wait()
        for bg in range(PAIRS // 8):
            rows = [row2 + (bg * 16 + 2 * p) for p in range(8)]

            def step(c, accs, rows=rows, slot=slot):
                chunk = gbuf.at[slot, c]
                return tuple(
                    accs[p] + plsc.load_gather(chunk, [rows[p], col8])
                    for p in range(8))

            accs = lax.fori_loop(0, LC, step, tuple(zero16
                                                    for _ in range(8)))
            for p in range(8):
                plsc.addupdate(out_v.at[pl.ds((bg * 8 + p) * 16, 16)],
                               accs[p])

    fire(0, 0)

    def body(i, _):
        g = i * 2
        fire(g + 1, 1)
        drain_accum(g, 0)

        @pl.when(g + 2 < NG)
        def _():
            fire(g + 2, 0)

        drain_accum(g + 1, 1)
        return 0

    lax.fori_loop(0, NG // 2, body, 0)
    for t in range(PAIRS):
        v = out_v[pl.ds(t * 16, 16)]
        out_v[pl.ds(t * 16, 16)] = v * (1.0 / SEQ) + bias
    pltpu.sync_copy(out_v,
                    out_hbm.at[pl.ds(base * DPAD, COLS_PER_W * DPAD)])


def kernel(x, table, W, b):
    xt = x.astype(jnp.int32).T
    wt = jnp.zeros((EMBED, DPAD), jnp.float32).at[:, :NCLS].set(W.T)
    bias = (jnp.zeros((16,), jnp.float32)
            .at[:NCLS].set(b).at[8:8 + NCLS].set(b))
    tw_t = _table_times_w(table.T, wt)
    tw = _sc_transpose(tw_t).reshape(VOCAB_P, DPAD)
    out = _sc_pool(tw, xt, bias)
    return out.reshape(BATCH, DPAD)[:, :NCLS]
